# Initial kernel scaffold; baseline (speedup 1.0000x reference)
import jax
import jax.numpy as jnp
from jax.experimental import pallas as pl


def kernel(X, table):
    raise NotImplementedError("write your pallas kernel here")


# speedup vs baseline: 3.2366x; 3.2366x over previous
import jax
import jax.numpy as jnp
from jax import lax
from jax.experimental import pallas as pl
from jax.experimental.pallas import tpu as pltpu
from jax.experimental.pallas import tpu_sc as plsc

N_ITEMS = 100000
D = 128
BATCH = 4096
HIST = 50
B = BATCH * HIST
NC = 2
NS = 16
NW = NC * NS
B_PER_W = B // NW
CHUNK = 128
N_CHUNKS = B_PER_W // CHUNK
IDX_ROWS = B // CHUNK


def _emb_body(x_hbm, tab_hbm, out_hbm, idx_v, buf_a, buf_b, sem_a, sem_b):
    wid = lax.axis_index("s") * NC + lax.axis_index("c")
    base = wid * B_PER_W

    pltpu.sync_copy(x_hbm.at[wid], idx_v)

    zeros16 = jnp.zeros((16,), jnp.float32)

    def fixup(buf, j):
        vmin = idx_v[j, pl.ds(0, 16)]
        for g in range(1, CHUNK // 16):
            vmin = jnp.minimum(vmin, idx_v[j, pl.ds(g * 16, 16)])
        m = vmin[0]
        for l in range(1, 16):
            m = jnp.minimum(m, vmin[l])

        @pl.when(m == 0)
        def _():
            def group_body(g, carry):
                iv = idx_v[j, pl.ds(g * 16, 16)]
                for l in range(16):
                    @pl.when(iv[l] == 0)
                    def _(l=l):
                        for cblk in range(D // 16):
                            buf[g * 16 + l, pl.ds(cblk * 16, 16)] = zeros16
                return carry

            lax.fori_loop(0, CHUNK // 16, group_body, 0)

    def gather(j, buf, sem):
        return pltpu.make_async_copy(tab_hbm.at[idx_v.at[j]], buf, sem)

    gather(0, buf_a, sem_a).start()

    def pair_body(p, carry):
        ja = 2 * p
        jb = ja + 1
        gather(ja, buf_a, sem_a).wait()
        gather(jb, buf_b, sem_b).start()
        fixup(buf_a, ja)
        pltpu.sync_copy(buf_a, out_hbm.at[pl.ds(base + ja * CHUNK, CHUNK)])
        gather(jb, buf_b, sem_b).wait()

        @pl.when(p < N_CHUNKS // 2 - 1)
        def _():
            gather(ja + 2, buf_a, sem_a).start()

        fixup(buf_b, jb)
        pltpu.sync_copy(buf_b, out_hbm.at[pl.ds(base + jb * CHUNK, CHUNK)])
        return carry

    lax.fori_loop(0, N_CHUNKS // 2, pair_body, 0)


def kernel(X, table):
    x2 = X.reshape(NW, N_CHUNKS, CHUNK)
    mesh = plsc.VectorSubcoreMesh(core_axis_name="c", subcore_axis_name="s")
    out = pl.kernel(
        _emb_body,
        out_type=jax.ShapeDtypeStruct((B, D), jnp.float32),
        mesh=mesh,
        scratch_types=[
            pltpu.VMEM((N_CHUNKS, CHUNK), jnp.int32),
            pltpu.VMEM((CHUNK, D), jnp.float32),
            pltpu.VMEM((CHUNK, D), jnp.float32),
            pltpu.SemaphoreType.DMA,
            pltpu.SemaphoreType.DMA,
        ],
    )(x2, table)
    return out.reshape(BATCH, HIST, D)


# speedup vs baseline: 6.0156x; 1.8586x over previous
import jax
import jax.numpy as jnp
from jax import lax
from jax.experimental import pallas as pl
from jax.experimental.pallas import tpu as pltpu
from jax.experimental.pallas import tpu_sc as plsc

N_ITEMS = 100000
D = 128
BATCH = 4096
HIST = 50
HP = 56
NC = 2
NS = 16
NW = NC * NS
B_PER_W = BATCH // NW
GB = 8
N_GROUPS = B_PER_W // GB
SLAB = B_PER_W * HP
OFFS = (0, 16, 32, 48)


def _emb_body(x_hbm, tab_hbm, out_hbm, idx_v, buf_a, buf_b, sem_a, sem_b):
    wid = lax.axis_index("s") * NC + lax.axis_index("c")
    row0 = wid * B_PER_W

    pltpu.sync_copy(x_hbm.at[pl.ds(wid * SLAB, SLAB)], idx_v)

    zeros16 = jnp.zeros((16,), jnp.float32)

    def gathers(b0, buf, sem):
        return [
            pltpu.make_async_copy(
                tab_hbm.at[idx_v.at[pl.ds((b0 + i) * HP, HIST)]],
                buf.at[i], sem)
            for i in range(GB)
        ]

    def start(b0, buf, sem):
        for c in gathers(b0, buf, sem):
            c.start()

    def drain(b0, buf, sem):
        for c in gathers(b0, buf, sem):
            c.wait()

    def fixup(buf, b0):
        def scan_batch(i, vmin):
            base = (b0 + i) * HP
            for o in OFFS:
                vmin = jnp.minimum(vmin, idx_v[pl.ds(base + o, 16)])
            return vmin

        vmin = lax.fori_loop(0, GB, scan_batch, jnp.ones((16,), jnp.int32))
        m = vmin[0]
        for l in range(1, 16):
            m = jnp.minimum(m, vmin[l])

        @pl.when(m == 0)
        def _():
            def fix_batch(i, carry):
                base = (b0 + i) * HP
                for o in OFFS:
                    v = idx_v[pl.ds(base + o, 16)]
                    for l in range(16):
                        if o + l < HIST:
                            @pl.when(v[l] == 0)
                            def _(h=o + l, i=i):
                                for cblk in range(D // 16):
                                    buf[i, h, pl.ds(cblk * 16, 16)] = zeros16
                return carry

            lax.fori_loop(0, GB, fix_batch, 0)

    start(0, buf_a, sem_a)

    def pair_body(p, carry):
        ga = 2 * p
        gb = ga + 1
        drain(ga * GB, buf_a, sem_a)
        start(gb * GB, buf_b, sem_b)
        fixup(buf_a, ga * GB)
        pltpu.sync_copy(buf_a, out_hbm.at[pl.ds(row0 + ga * GB, GB)])
        drain(gb * GB, buf_b, sem_b)

        @pl.when(p < N_GROUPS // 2 - 1)
        def _():
            start((ga + 2) * GB, buf_a, sem_a)

        fixup(buf_b, gb * GB)
        pltpu.sync_copy(buf_b, out_hbm.at[pl.ds(row0 + gb * GB, GB)])
        return carry

    lax.fori_loop(0, N_GROUPS // 2, pair_body, 0)


def kernel(X, table):
    xp = jnp.pad(X, ((0, 0), (0, HP - HIST)), constant_values=1)
    xp = xp.reshape(BATCH * HP)
    mesh = plsc.VectorSubcoreMesh(core_axis_name="c", subcore_axis_name="s")
    return pl.kernel(
        _emb_body,
        out_type=jax.ShapeDtypeStruct((BATCH, HIST, D), jnp.float32),
        mesh=mesh,
        scratch_types=[
            pltpu.VMEM((SLAB,), jnp.int32),
            pltpu.VMEM((GB, HIST, D), jnp.float32),
            pltpu.VMEM((GB, HIST, D), jnp.float32),
            pltpu.SemaphoreType.DMA,
            pltpu.SemaphoreType.DMA,
        ],
    )(xp, table)


# speedup vs baseline: 6.0322x; 1.0028x over previous
import jax
import jax.numpy as jnp
from jax import lax
from jax.experimental import pallas as pl
from jax.experimental.pallas import tpu as pltpu
from jax.experimental.pallas import tpu_sc as plsc

N_ITEMS = 100000
D = 128
BATCH = 4096
HIST = 50
HP = 56
NC = 2
NS = 16
NW = NC * NS
B_PER_W = BATCH // NW
GB = 8
N_GROUPS = B_PER_W // GB
SLAB = B_PER_W * HP
OFFS = (0, 16, 32, 48)


def _emb_body(x_hbm, tab_hbm, out_hbm, idx_v, buf_a, buf_b, sem_a, sem_b):
    wid = lax.axis_index("s") * NC + lax.axis_index("c")
    row0 = wid * B_PER_W

    pltpu.sync_copy(x_hbm.at[pl.ds(wid * SLAB, SLAB)], idx_v)

    zeros16 = jnp.zeros((16,), jnp.float32)

    def gathers(b0, buf, sem):
        return [
            pltpu.make_async_copy(
                tab_hbm.at[idx_v.at[pl.ds((b0 + i) * HP, HIST)]],
                buf.at[i], sem)
            for i in range(GB)
        ]

    def start(b0, buf, sem):
        for c in gathers(b0, buf, sem):
            c.start()

    def drain(b0, buf, sem):
        for c in gathers(b0, buf, sem):
            c.wait()

    def fixup(buf, b0):
        def scan_batch(i, vmin):
            base = (b0 + i) * HP
            for o in OFFS:
                vmin = jnp.minimum(vmin, idx_v[pl.ds(base + o, 16)])
            return vmin

        vmin = lax.fori_loop(0, GB, scan_batch, jnp.ones((16,), jnp.int32))
        m = vmin[0]
        for l in range(1, 16):
            m = jnp.minimum(m, vmin[l])

        @pl.when(m == 0)
        def _():
            def fix_batch(i, carry):
                base = (b0 + i) * HP
                for o in OFFS:
                    v = idx_v[pl.ds(base + o, 16)]
                    for l in range(16):
                        if o + l < HIST:
                            @pl.when(v[l] == 0)
                            def _(h=o + l, i=i):
                                for cblk in range(D // 16):
                                    buf[i, h, pl.ds(cblk * 16, 16)] = zeros16
                return carry

            lax.fori_loop(0, GB, fix_batch, 0)

    start(0, buf_a, sem_a)

    def pair_body(p, carry):
        ga = 2 * p
        gb = ga + 1
        drain(ga * GB, buf_a, sem_a)
        start(gb * GB, buf_b, sem_b)
        fixup(buf_a, ga * GB)
        pltpu.sync_copy(buf_a, out_hbm.at[pl.ds(row0 + ga * GB, GB)])
        drain(gb * GB, buf_b, sem_b)

        @pl.when(p < N_GROUPS // 2 - 1)
        def _():
            start((ga + 2) * GB, buf_a, sem_a)

        fixup(buf_b, gb * GB)
        pltpu.sync_copy(buf_b, out_hbm.at[pl.ds(row0 + gb * GB, GB)])
        return carry

    lax.fori_loop(0, N_GROUPS // 2, pair_body, 0)


def kernel(X, table):
    xp = jnp.pad(X, ((0, 0), (0, HP - HIST)), constant_values=1)
    xp = xp.reshape(BATCH * HP)
    mesh = plsc.VectorSubcoreMesh(core_axis_name="c", subcore_axis_name="s")
    return pl.kernel(
        _emb_body,
        out_type=jax.ShapeDtypeStruct((BATCH, HIST, D), jnp.float32),
        mesh=mesh,
        compiler_params=pltpu.CompilerParams(use_tc_tiling_on_sc=True),
        scratch_types=[
            pltpu.VMEM((SLAB,), jnp.int32),
            pltpu.VMEM((GB, HIST, D), jnp.float32),
            pltpu.VMEM((GB, HIST, D), jnp.float32),
            pltpu.SemaphoreType.DMA,
            pltpu.SemaphoreType.DMA,
        ],
    )(xp, table)
